# baseline (speedup 1.0000x reference)
import functools

import jax
import jax.numpy as jnp
from jax import lax
from jax.experimental import pallas as pl
from jax.experimental.pallas import tpu as pltpu
from jax.experimental.pallas import tpu_sc as plsc

_VOCAB = 1_000_000
_EMBED = 64
_L = 200
_B = 4096
_RT_W = 16
_ROW_W = 128
_NC = 2
_NS = 16
_NW = _NC * _NS
_BPW = _B // _NW
_TCHUNK = 10
_NCHUNK = _L // _TCHUNK

_MM_CB = 32768


def _rt_body(tt_ref, ws_ref, bs_ref, o_ref):
    r = jnp.dot(ws_ref[...], tt_ref[...], preferred_element_type=jnp.float32,
                precision=lax.Precision.HIGHEST)
    r = r + bs_ref[...]
    payload = r.T.astype(jnp.bfloat16)
    zeros = jnp.zeros((_MM_CB, _ROW_W - _RT_W), jnp.bfloat16)
    o_ref[...] = jnp.concatenate([payload, zeros], axis=1)


def _reduce_table(table_t, w_scaled_t, b_scaled):
    grid = (_VOCAB + _MM_CB - 1) // _MM_CB
    return pl.pallas_call(
        _rt_body,
        grid=(grid,),
        in_specs=[
            pl.BlockSpec((_EMBED, _MM_CB), lambda i: (0, i)),
            pl.BlockSpec((_RT_W, _EMBED), lambda i: (0, 0)),
            pl.BlockSpec((_RT_W, 1), lambda i: (0, 0)),
        ],
        out_specs=pl.BlockSpec((_MM_CB, _ROW_W), lambda i: (i, 0)),
        out_shape=jax.ShapeDtypeStruct((_VOCAB, _ROW_W), jnp.bfloat16),
    )(table_t, w_scaled_t, b_scaled)


def _sc_pool(rt_rows, text):
    mesh = plsc.VectorSubcoreMesh(core_axis_name="c", subcore_axis_name="s")

    @functools.partial(
        pl.kernel,
        mesh=mesh,
        out_type=jax.ShapeDtypeStruct((_B, 2 * _RT_W), jnp.float32),
        scratch_types=[
            pltpu.VMEM((_L, _BPW), jnp.int32),
            pltpu.VMEM((2, _TCHUNK, _BPW, 2 * _RT_W), jnp.bfloat16),
            pltpu.VMEM((_BPW, 2 * _RT_W), jnp.float32),
            pltpu.SemaphoreType.DMA,
            pltpu.SemaphoreType.DMA,
        ],
        compiler_params=pltpu.CompilerParams(
            use_tc_tiling_on_sc=False, needs_layout_passes=False),
    )
    def k(rt_hbm, text_hbm, out_hbm, idx_v, rows_v, acc_v, sem0, sem1):
        wid = lax.axis_index("s") * _NC + lax.axis_index("c")
        base = wid * _BPW

        pltpu.sync_copy(text_hbm.at[:, pl.ds(base, _BPW)], idx_v)

        def scale_body(t, carry):
            for q in range(_BPW // 16):
                s = idx_v[t, pl.ds(16 * q, 16)]
                idx_v[t, pl.ds(16 * q, 16)] = s * 4
            return carry

        lax.fori_loop(0, _L, scale_body, 0)

        def issue(chunk, p, sem):
            for j in range(_TCHUNK):
                pltpu.async_copy(
                    rt_hbm.at[idx_v.at[chunk * _TCHUNK + j]],
                    rows_v.at[p, j],
                    sem,
                )

        def drain(chunk, p, sem):
            for j in range(_TCHUNK):
                pltpu.make_async_copy(
                    rt_hbm.at[idx_v.at[chunk * _TCHUNK + j]],
                    rows_v.at[p, j],
                    sem,
                ).wait()

        def accumulate(p):
            def body(b, carry):
                alo = acc_v[b, pl.ds(0, _RT_W)]
                ahi = acc_v[b, pl.ds(_RT_W, _RT_W)]
                for j in range(_TCHUNK):
                    w32 = plsc.bitcast(rows_v[p, j, b, :], jnp.int32)
                    lo = lax.bitcast_convert_type(
                        lax.shift_left(w32, 16), jnp.float32)
                    hi = lax.bitcast_convert_type(
                        lax.shift_left(lax.shift_right_logical(w32, 16), 16),
                        jnp.float32)
                    alo = alo + lo
                    ahi = ahi + hi
                acc_v[b, pl.ds(0, _RT_W)] = alo
                acc_v[b, pl.ds(_RT_W, _RT_W)] = ahi
                return carry

            lax.fori_loop(0, _BPW, body, 0)

        def zero_acc():
            z = jnp.zeros((_RT_W,), jnp.float32)

            def body(b, carry):
                acc_v[b, pl.ds(0, _RT_W)] = z
                acc_v[b, pl.ds(_RT_W, _RT_W)] = z
                return carry

            lax.fori_loop(0, _BPW, body, 0)

        zero_acc()
        issue(0, 0, sem0)
        issue(1, 1, sem1)

        def body(g2, carry):
            g = g2 * 2
            drain(g, 0, sem0)
            accumulate(0)
            issue(g + 2, 0, sem0)
            drain(g + 1, 1, sem1)
            accumulate(1)
            issue(g + 3, 1, sem1)
            return carry

        lax.fori_loop(0, (_NCHUNK - 2) // 2, body, 0)

        drain(_NCHUNK - 2, 0, sem0)
        accumulate(0)
        drain(_NCHUNK - 1, 1, sem1)
        accumulate(1)

        pltpu.sync_copy(acc_v, out_hbm.at[pl.ds(base, _BPW)])

    return k(rt_rows, text)


def kernel(text, text_lengths, table, W, b):
    del text_lengths
    out_dim = W.shape[1]
    inv_l = 1.0 / _L
    ws_t = (
        jnp.zeros((_RT_W, _EMBED), jnp.float32)
        .at[:out_dim, :]
        .set(W.astype(jnp.float32).T * inv_l)
    )
    bs = (
        jnp.zeros((_RT_W, 1), jnp.float32)
        .at[:out_dim, 0]
        .set(b.astype(jnp.float32) * inv_l)
    )
    rt = _reduce_table(table.T, ws_t, bs)
    rt_rows = rt.reshape(_VOCAB * 4, 32)
    pooled = _sc_pool(rt_rows, text.astype(jnp.int32))
    cols = [16 * (j % 2) + j // 2 for j in range(out_dim)]
    return jnp.concatenate([pooled[:, c:c + 1] for c in cols], axis=1)


# speedup vs baseline: 3.5244x; 3.5244x over previous
import functools

import jax
import jax.numpy as jnp
from jax import lax
from jax.experimental import pallas as pl
from jax.experimental.pallas import tpu as pltpu
from jax.experimental.pallas import tpu_sc as plsc

_VOCAB = 1_000_000
_EMBED = 64
_L = 200
_B = 4096
_RT_W = 16
_ROW_W = 128
_NC = 2
_NS = 16
_NW = _NC * _NS
_BPW = _B // _NW
_TCHUNK = 20
_NCHUNK = _L // _TCHUNK

_MM_CB = 32768


def _rt_body(tt_ref, ws_ref, bs_ref, o_ref):
    r = jnp.dot(ws_ref[...], tt_ref[...], preferred_element_type=jnp.float32,
                precision=lax.Precision.HIGHEST)
    r = r + bs_ref[...]
    o_ref[:, :_RT_W] = r.T


def _reduce_table(table_t, w_scaled_t, b_scaled):
    grid = (_VOCAB + _MM_CB - 1) // _MM_CB
    return pl.pallas_call(
        _rt_body,
        grid=(grid,),
        in_specs=[
            pl.BlockSpec((_EMBED, _MM_CB), lambda i: (0, i)),
            pl.BlockSpec((_RT_W, _EMBED), lambda i: (0, 0)),
            pl.BlockSpec((_RT_W, 1), lambda i: (0, 0)),
        ],
        out_specs=pl.BlockSpec((_MM_CB, _ROW_W), lambda i: (i, 0)),
        out_shape=jax.ShapeDtypeStruct((_VOCAB, _ROW_W), jnp.float32),
    )(table_t, w_scaled_t, b_scaled)


def _sc_pool(rt_rows, text):
    mesh = plsc.VectorSubcoreMesh(core_axis_name="c", subcore_axis_name="s")

    @functools.partial(
        pl.kernel,
        mesh=mesh,
        out_type=jax.ShapeDtypeStruct((_B, _RT_W), jnp.float32),
        scratch_types=[
            pltpu.VMEM((_L, _BPW), jnp.int32),
            pltpu.VMEM((2, _TCHUNK, _BPW, _RT_W), jnp.float32),
            pltpu.VMEM((_BPW, _RT_W), jnp.float32),
            pltpu.SemaphoreType.DMA,
            pltpu.SemaphoreType.DMA,
        ],
        compiler_params=pltpu.CompilerParams(use_tc_tiling_on_sc=False),
    )
    def k(rt_hbm, text_hbm, out_hbm, idx_v, rows_v, acc_v, sem0, sem1):
        wid = lax.axis_index("s") * _NC + lax.axis_index("c")
        base = wid * _BPW

        pltpu.sync_copy(text_hbm.at[:, pl.ds(base, _BPW)], idx_v)

        def scale_body(t, carry):
            for q in range(_BPW // 16):
                s = idx_v[t, pl.ds(16 * q, 16)]
                idx_v[t, pl.ds(16 * q, 16)] = s * 8
            return carry

        lax.fori_loop(0, _L, scale_body, 0)

        def issue(chunk, p, sem):
            for j in range(_TCHUNK):
                pltpu.async_copy(
                    rt_hbm.at[idx_v.at[chunk * _TCHUNK + j]],
                    rows_v.at[p, j],
                    sem,
                )

        def drain(chunk, p, sem):
            for j in range(_TCHUNK):
                pltpu.make_async_copy(
                    rt_hbm.at[idx_v.at[chunk * _TCHUNK + j]],
                    rows_v.at[p, j],
                    sem,
                ).wait()

        def accumulate(p):
            def body(b, carry):
                a = acc_v[b, :]
                for j in range(_TCHUNK):
                    a = a + rows_v[p, j, b, :]
                acc_v[b, :] = a
                return carry

            lax.fori_loop(0, _BPW, body, 0)

        def zero_acc():
            z = jnp.zeros((_RT_W,), jnp.float32)

            def body(b, carry):
                acc_v[b, :] = z
                return carry

            lax.fori_loop(0, _BPW, body, 0)

        zero_acc()
        issue(0, 0, sem0)
        issue(1, 1, sem1)

        def body(g2, carry):
            g = g2 * 2
            drain(g, 0, sem0)
            accumulate(0)
            issue(g + 2, 0, sem0)
            drain(g + 1, 1, sem1)
            accumulate(1)
            issue(g + 3, 1, sem1)
            return carry

        lax.fori_loop(0, (_NCHUNK - 2) // 2, body, 0)

        drain(_NCHUNK - 2, 0, sem0)
        accumulate(0)
        drain(_NCHUNK - 1, 1, sem1)
        accumulate(1)

        pltpu.sync_copy(acc_v, out_hbm.at[pl.ds(base, _BPW)])

    return k(rt_rows, text)


def kernel(text, text_lengths, table, W, b):
    del text_lengths
    out_dim = W.shape[1]
    inv_l = 1.0 / _L
    ws_t = (
        jnp.zeros((_RT_W, _EMBED), jnp.float32)
        .at[:out_dim, :]
        .set(W.astype(jnp.float32).T * inv_l)
    )
    bs = (
        jnp.zeros((_RT_W, 1), jnp.float32)
        .at[:out_dim, 0]
        .set(b.astype(jnp.float32) * inv_l)
    )
    rt = _reduce_table(table.T, ws_t, bs)
    pooled = _sc_pool(rt.reshape(_VOCAB * 8, _RT_W), text.astype(jnp.int32))
    return pooled[:, :out_dim]
